# baseline (speedup 1.0000x reference)
import functools

import jax
import jax.numpy as jnp
from jax import lax
from jax.experimental import pallas as pl
from jax.experimental.pallas import tpu as pltpu
from jax.experimental.pallas import tpu_sc as plsc

N = 10000
E = 320000
D = 128

NC = 2
NS = 16
NW = NC * NS

K = 50
NCHUNK = 200
G = 10
NG = NCHUNK // G
EPW = NCHUNK * K
NP = 10112
ROWS_PER_S = NP // NS


def _sc_scatter_build():
    mesh = plsc.VectorSubcoreMesh(core_axis_name="c", subcore_axis_name="s")

    @functools.partial(
        pl.kernel,
        out_type=jax.ShapeDtypeStruct((NC, NP, D), jnp.float32),
        mesh=mesh,
        scratch_types=[
            pltpu.VMEM((2, G, K), jnp.int32),
            pltpu.VMEM((2, G, K), jnp.int32),
            pltpu.VMEM((2, K, D), jnp.float32),
            pltpu.VMEM_SHARED((NP, D), jnp.float32),
            pltpu.SemaphoreType.DMA,
            pltpu.SemaphoreType.DMA,
            pltpu.SemaphoreType.DMA,
            pltpu.SemaphoreType.DMA,
        ],
    )
    def sc_scatter(x_hbm, row_hbm, col_hbm, zeros_hbm, out_hbm,
                   rowg, colg, rows_v, acc, semg0, semg1, semi0, semi1):
        c = lax.axis_index("c")
        s = lax.axis_index("s")
        wid = s * NC + c
        semg = (semg0, semg1)
        semi = (semi0, semi1)

        pltpu.sync_copy(zeros_hbm, acc.at[pl.ds(s * ROWS_PER_S, ROWS_PER_S)])

        pltpu.sync_copy(row_hbm.at[wid, 0], rowg.at[0])
        pltpu.sync_copy(col_hbm.at[wid, 0], colg.at[0])
        pltpu.async_copy(row_hbm.at[wid, 1], rowg.at[1], semi1)
        pltpu.async_copy(col_hbm.at[wid, 1], colg.at[1], semi1)

        plsc.subcore_barrier()

        def start_g(ip, lt, b):
            pltpu.async_copy(x_hbm.at[rowg.at[ip, lt]], rows_v.at[b], semg[b])

        def wait_g(ip, lt, b):
            pltpu.make_async_copy(x_hbm.at[rowg.at[ip, lt]], rows_v.at[b],
                                  semg[b]).wait()

        def scat(ip, lt, b):
            pltpu.sync_copy(rows_v.at[b], acc.at[colg.at[ip, lt]], add=True)

        start_g(0, 0, 0)
        start_g(0, 1, 1)

        def ubody(u, carry):
            for p in (0, 1):
                g = 2 * u + p

                def pair(t, cc):
                    for q in (0, 1):
                        lt = 2 * t + q
                        wait_g(p, lt, q)
                        scat(p, lt, q)
                        start_g(p, lt + 2, q)
                    return cc

                lax.fori_loop(0, G // 2 - 1, pair, 0)

                wait_g(p, G - 2, 0)
                scat(p, G - 2, 0)
                wait_g(p, G - 1, 1)
                scat(p, G - 1, 1)

                @pl.when(g + 2 < NG)
                def _():
                    pltpu.async_copy(row_hbm.at[wid, g + 2], rowg.at[p],
                                     semi[p])
                    pltpu.async_copy(col_hbm.at[wid, g + 2], colg.at[p],
                                     semi[p])

                @pl.when(g + 1 < NG)
                def _():
                    pltpu.make_async_copy(row_hbm.at[wid, g + 1],
                                          rowg.at[1 - p], semi[1 - p]).wait()
                    pltpu.make_async_copy(col_hbm.at[wid, g + 1],
                                          colg.at[1 - p], semi[1 - p]).wait()
                    start_g(1 - p, 0, 0)
                    start_g(1 - p, 1, 1)
            return carry

        lax.fori_loop(0, NG // 2, ubody, 0)

        plsc.subcore_barrier()

        pltpu.sync_copy(acc.at[pl.ds(s * ROWS_PER_S, ROWS_PER_S)],
                        out_hbm.at[c, pl.ds(s * ROWS_PER_S, ROWS_PER_S)])

    return sc_scatter


_sc_scatter = _sc_scatter_build()


BN = 2000
dn = (((1,), (1,)), ((), ()))


def _tc_finish_body(part_ref, x_ref, wroot_ref, b_ref, wrel_ref, out_ref):
    agg = part_ref[0] + part_ref[1]
    rel = lax.dot_general(agg, wrel_ref[...], dn,
                          preferred_element_type=jnp.float32)
    root = lax.dot_general(x_ref[...], wroot_ref[...], dn,
                           preferred_element_type=jnp.float32)
    out_ref[...] = jnp.maximum(rel + root + b_ref[...], 0.0)


def _tc_finish(part, x, W_root, b_root, W_rel):
    return pl.pallas_call(
        _tc_finish_body,
        grid=(N // BN,),
        in_specs=[
            pl.BlockSpec((NC, BN, D), lambda i: (0, i, 0)),
            pl.BlockSpec((BN, D), lambda i: (i, 0)),
            pl.BlockSpec((D, D), lambda i: (0, 0)),
            pl.BlockSpec((1, D), lambda i: (0, 0)),
            pl.BlockSpec((D, D), lambda i: (0, 0)),
        ],
        out_specs=pl.BlockSpec((BN, D), lambda i: (i, 0)),
        out_shape=jax.ShapeDtypeStruct((N, D), jnp.float32),
    )(part, x, W_root, b_root.reshape(1, D), W_rel)


def kernel(x, row, col, batch, W_root, b_root, W_rel):
    row3 = row.astype(jnp.int32).reshape(NW, NG, G, K)
    col3 = col.astype(jnp.int32).reshape(NW, NG, G, K)
    zeros = jnp.zeros((ROWS_PER_S, D), jnp.float32)
    part = _sc_scatter(x, row3, col3, zeros)
    return _tc_finish(part, x, W_root, b_root, W_rel)


# speedup vs baseline: 1.2109x; 1.2109x over previous
import functools

import jax
import jax.numpy as jnp
from jax import lax
from jax.experimental import pallas as pl
from jax.experimental.pallas import tpu as pltpu
from jax.experimental.pallas import tpu_sc as plsc

N = 10000
E = 320000
D = 128

NC = 2
NS = 16
NW = NC * NS

K = 100
NCHUNK = 100
G = 10
NG = NCHUNK // G
EPW = NCHUNK * K
NP = 10112
ROWS_PER_S = NP // NS


def _sc_scatter_build():
    mesh = plsc.VectorSubcoreMesh(core_axis_name="c", subcore_axis_name="s")

    @functools.partial(
        pl.kernel,
        out_type=jax.ShapeDtypeStruct((NC, NP, D), jnp.float32),
        mesh=mesh,
        scratch_types=[
            pltpu.VMEM((2, G, K), jnp.int32),
            pltpu.VMEM((2, G, K), jnp.int32),
            pltpu.VMEM((2, K, D), jnp.float32),
            pltpu.VMEM_SHARED((NP, D), jnp.float32),
            pltpu.SemaphoreType.DMA,
            pltpu.SemaphoreType.DMA,
            pltpu.SemaphoreType.DMA,
            pltpu.SemaphoreType.DMA,
        ],
    )
    def sc_scatter(x_hbm, row_hbm, col_hbm, zeros_hbm, out_hbm,
                   rowg, colg, rows_v, acc, semg0, semg1, semi0, semi1):
        c = lax.axis_index("c")
        s = lax.axis_index("s")
        wid = s * NC + c
        semg = (semg0, semg1)
        semi = (semi0, semi1)

        pltpu.sync_copy(zeros_hbm, acc.at[pl.ds(s * ROWS_PER_S, ROWS_PER_S)])

        pltpu.sync_copy(row_hbm.at[wid, 0], rowg.at[0])
        pltpu.sync_copy(col_hbm.at[wid, 0], colg.at[0])
        pltpu.async_copy(row_hbm.at[wid, 1], rowg.at[1], semi1)
        pltpu.async_copy(col_hbm.at[wid, 1], colg.at[1], semi1)

        plsc.subcore_barrier()

        def start_g(ip, lt, b):
            pltpu.async_copy(x_hbm.at[rowg.at[ip, lt]], rows_v.at[b], semg[b])

        def wait_g(ip, lt, b):
            pltpu.make_async_copy(x_hbm.at[rowg.at[ip, lt]], rows_v.at[b],
                                  semg[b]).wait()

        def scat(ip, lt, b):
            pltpu.sync_copy(rows_v.at[b], acc.at[colg.at[ip, lt]], add=True)

        start_g(0, 0, 0)
        start_g(0, 1, 1)

        def ubody(u, carry):
            for p in (0, 1):
                g = 2 * u + p

                def pair(t, cc):
                    for q in (0, 1):
                        lt = 2 * t + q
                        wait_g(p, lt, q)
                        scat(p, lt, q)
                        start_g(p, lt + 2, q)
                    return cc

                lax.fori_loop(0, G // 2 - 1, pair, 0)

                wait_g(p, G - 2, 0)
                scat(p, G - 2, 0)
                wait_g(p, G - 1, 1)
                scat(p, G - 1, 1)

                @pl.when(g + 2 < NG)
                def _():
                    pltpu.async_copy(row_hbm.at[wid, g + 2], rowg.at[p],
                                     semi[p])
                    pltpu.async_copy(col_hbm.at[wid, g + 2], colg.at[p],
                                     semi[p])

                @pl.when(g + 1 < NG)
                def _():
                    pltpu.make_async_copy(row_hbm.at[wid, g + 1],
                                          rowg.at[1 - p], semi[1 - p]).wait()
                    pltpu.make_async_copy(col_hbm.at[wid, g + 1],
                                          colg.at[1 - p], semi[1 - p]).wait()
                    start_g(1 - p, 0, 0)
                    start_g(1 - p, 1, 1)
            return carry

        lax.fori_loop(0, NG // 2, ubody, 0)

        plsc.subcore_barrier()

        pltpu.sync_copy(acc.at[pl.ds(s * ROWS_PER_S, ROWS_PER_S)],
                        out_hbm.at[c, pl.ds(s * ROWS_PER_S, ROWS_PER_S)])

    return sc_scatter


_sc_scatter = _sc_scatter_build()


BN = 2000
dn = (((1,), (1,)), ((), ()))


def _tc_finish_body(part_ref, x_ref, wroot_ref, b_ref, wrel_ref, out_ref):
    agg = part_ref[0] + part_ref[1]
    rel = lax.dot_general(agg, wrel_ref[...], dn,
                          preferred_element_type=jnp.float32)
    root = lax.dot_general(x_ref[...], wroot_ref[...], dn,
                           preferred_element_type=jnp.float32)
    out_ref[...] = jnp.maximum(rel + root + b_ref[...], 0.0)


def _tc_finish(part, x, W_root, b_root, W_rel):
    return pl.pallas_call(
        _tc_finish_body,
        grid=(N // BN,),
        in_specs=[
            pl.BlockSpec((NC, BN, D), lambda i: (0, i, 0)),
            pl.BlockSpec((BN, D), lambda i: (i, 0)),
            pl.BlockSpec((D, D), lambda i: (0, 0)),
            pl.BlockSpec((1, D), lambda i: (0, 0)),
            pl.BlockSpec((D, D), lambda i: (0, 0)),
        ],
        out_specs=pl.BlockSpec((BN, D), lambda i: (i, 0)),
        out_shape=jax.ShapeDtypeStruct((N, D), jnp.float32),
    )(part, x, W_root, b_root.reshape(1, D), W_rel)


def kernel(x, row, col, batch, W_root, b_root, W_rel):
    row3 = row.astype(jnp.int32).reshape(NW, NG, G, K)
    col3 = col.astype(jnp.int32).reshape(NW, NG, G, K)
    zeros = jnp.zeros((ROWS_PER_S, D), jnp.float32)
    part = _sc_scatter(x, row3, col3, zeros)
    return _tc_finish(part, x, W_root, b_root, W_rel)


# speedup vs baseline: 1.2188x; 1.0065x over previous
import functools

import jax
import jax.numpy as jnp
from jax import lax
from jax.experimental import pallas as pl
from jax.experimental.pallas import tpu as pltpu
from jax.experimental.pallas import tpu_sc as plsc

N = 10000
E = 320000
D = 128

NC = 2
NS = 16
NW = NC * NS

K = 100
NCHUNK = 100
G = 10
NG = NCHUNK // G
EPW = NCHUNK * K
NP = 10112
ROWS_PER_S = NP // NS


def _sc_scatter_build():
    mesh = plsc.VectorSubcoreMesh(core_axis_name="c", subcore_axis_name="s")

    @functools.partial(
        pl.kernel,
        out_type=jax.ShapeDtypeStruct((NC, NP, D), jnp.float32),
        mesh=mesh,
        scratch_types=[
            pltpu.VMEM((2, G, K), jnp.int32),
            pltpu.VMEM((2, G, K), jnp.int32),
            pltpu.VMEM((2, K, D), jnp.float32),
            pltpu.VMEM_SHARED((NP, D), jnp.float32),
            pltpu.SemaphoreType.DMA,
            pltpu.SemaphoreType.DMA,
            pltpu.SemaphoreType.DMA,
            pltpu.SemaphoreType.DMA,
        ],
    )
    def sc_scatter(x_hbm, row_hbm, col_hbm, zeros_hbm, out_hbm,
                   rowg, colg, rows_v, acc, semg0, semg1, semi0, semi1):
        c = lax.axis_index("c")
        s = lax.axis_index("s")
        wid = s * NC + c
        semg = (semg0, semg1)
        semi = (semi0, semi1)

        pltpu.sync_copy(zeros_hbm, acc.at[pl.ds(s * ROWS_PER_S, ROWS_PER_S)])

        pltpu.sync_copy(row_hbm.at[wid, 0], rowg.at[0])
        pltpu.sync_copy(col_hbm.at[wid, 0], colg.at[0])
        pltpu.async_copy(row_hbm.at[wid, 1], rowg.at[1], semi1)
        pltpu.async_copy(col_hbm.at[wid, 1], colg.at[1], semi1)

        plsc.subcore_barrier()

        def start_g(ip, lt, b):
            pltpu.async_copy(x_hbm.at[rowg.at[ip, lt]], rows_v.at[b], semg[b])

        def wait_g(ip, lt, b):
            pltpu.make_async_copy(x_hbm.at[rowg.at[ip, lt]], rows_v.at[b],
                                  semg[b]).wait()

        def scat(ip, lt, b):
            pltpu.sync_copy(rows_v.at[b], acc.at[colg.at[ip, lt]], add=True)

        start_g(0, 0, 0)
        start_g(0, 1, 1)

        def ubody(u, carry):
            for p in (0, 1):
                g = 2 * u + p

                def pair(t, cc):
                    for q in (0, 1):
                        lt = 2 * t + q
                        wait_g(p, lt, q)
                        scat(p, lt, q)
                        start_g(p, lt + 2, q)
                    return cc

                lax.fori_loop(0, G // 2 - 1, pair, 0)

                wait_g(p, G - 2, 0)
                scat(p, G - 2, 0)
                wait_g(p, G - 1, 1)
                scat(p, G - 1, 1)

                @pl.when(g + 2 < NG)
                def _():
                    pltpu.async_copy(row_hbm.at[wid, g + 2], rowg.at[p],
                                     semi[p])
                    pltpu.async_copy(col_hbm.at[wid, g + 2], colg.at[p],
                                     semi[p])

                @pl.when(g + 1 < NG)
                def _():
                    pltpu.make_async_copy(row_hbm.at[wid, g + 1],
                                          rowg.at[1 - p], semi[1 - p]).wait()
                    pltpu.make_async_copy(col_hbm.at[wid, g + 1],
                                          colg.at[1 - p], semi[1 - p]).wait()
                    start_g(1 - p, 0, 0)
                    start_g(1 - p, 1, 1)
            return carry

        lax.fori_loop(0, NG // 2, ubody, 0)

        plsc.subcore_barrier()

        pltpu.sync_copy(acc.at[pl.ds(s * ROWS_PER_S, ROWS_PER_S)],
                        out_hbm.at[c, pl.ds(s * ROWS_PER_S, ROWS_PER_S)])

    return sc_scatter


_sc_scatter = _sc_scatter_build()


BN = 5000
dn = (((1,), (1,)), ((), ()))


def _tc_finish_body(part_ref, x_ref, wroot_ref, b_ref, wrel_ref, out_ref):
    agg = part_ref[0] + part_ref[1]
    rel = lax.dot_general(agg, wrel_ref[...], dn,
                          preferred_element_type=jnp.float32)
    root = lax.dot_general(x_ref[...], wroot_ref[...], dn,
                           preferred_element_type=jnp.float32)
    out_ref[...] = jnp.maximum(rel + root + b_ref[...], 0.0)


def _tc_finish(part, x, W_root, b_root, W_rel):
    return pl.pallas_call(
        _tc_finish_body,
        grid=(N // BN,),
        in_specs=[
            pl.BlockSpec((NC, BN, D), lambda i: (0, i, 0)),
            pl.BlockSpec((BN, D), lambda i: (i, 0)),
            pl.BlockSpec((D, D), lambda i: (0, 0)),
            pl.BlockSpec((1, D), lambda i: (0, 0)),
            pl.BlockSpec((D, D), lambda i: (0, 0)),
        ],
        out_specs=pl.BlockSpec((BN, D), lambda i: (i, 0)),
        out_shape=jax.ShapeDtypeStruct((N, D), jnp.float32),
    )(part, x, W_root, b_root.reshape(1, D), W_rel)


def kernel(x, row, col, batch, W_root, b_root, W_rel):
    row3 = row.astype(jnp.int32).reshape(NW, NG, G, K)
    col3 = col.astype(jnp.int32).reshape(NW, NG, G, K)
    zeros = jnp.zeros((ROWS_PER_S, D), jnp.float32)
    part = _sc_scatter(x, row3, col3, zeros)
    return _tc_finish(part, x, W_root, b_root, W_rel)
